# baseline (speedup 1.0000x reference)
import functools

import jax
import jax.numpy as jnp
from jax import lax
from jax.experimental import pallas as pl
from jax.experimental.pallas import tpu as pltpu
from jax.experimental.pallas import tpu_sc as plsc

_N = 10000
_E = 320000
_NC = 2
_NS = 16
_NW = _NC * _NS
_EPW = _E // _NW
_CHUNK = 128
_NCH = 80
_EPWP = _CHUNK * _NCH
_NBUF = 2
_IDXH = 40
_NGRP = _NCH // _NBUF
_TRASH = 10100
_NA = _NS * 632
_RPS = _NA // _NS
_NP = _NS * 640
_DPS = _NP // _NS

_ROW_BLK = 1000


def _make_mesh():
    return plsc.VectorSubcoreMesh(core_axis_name="c", subcore_axis_name="s")


@functools.partial(
    pl.kernel,
    mesh=_make_mesh(),
    out_type=jax.ShapeDtypeStruct((_NC * _NP,), jnp.float32),
    scratch_types=[
        pltpu.VMEM((_NCH, _CHUNK), jnp.int32),
        pltpu.VMEM((_CHUNK,), jnp.float32),
        pltpu.VMEM((_DPS,), jnp.float32),
        pltpu.VMEM_SHARED((_NP,), jnp.float32),
    ],
)
def _deg_sc(dstp_hbm, out_hbm, dst_all, ones_v, stage_v, dacc):
    c = lax.axis_index("c")
    s = lax.axis_index("s")
    wid = c * _NS + s
    r0 = s * _DPS

    def zbody(k, carry):
        stage_v[pl.ds(k * 16, 16)] = jnp.zeros((16,), jnp.float32)
        return carry

    lax.fori_loop(0, _DPS // 16, zbody, 0)
    pltpu.sync_copy(stage_v, dacc.at[pl.ds(r0, _DPS)])
    for k in range(_CHUNK // 16):
        ones_v[pl.ds(k * 16, 16)] = jnp.full((16,), 1.0, jnp.float32)
    pltpu.sync_copy(dstp_hbm.at[wid], dst_all)
    plsc.subcore_barrier()

    def body(j, carry):
        pltpu.sync_copy(ones_v, dacc.at[dst_all.at[j]], add=True)
        return carry

    lax.fori_loop(0, _NCH, body, 0)
    plsc.subcore_barrier()
    pltpu.sync_copy(dacc.at[pl.ds(r0, _DPS)], stage_v)
    pltpu.sync_copy(stage_v, out_hbm.at[pl.ds(c * _NP + r0, _DPS)])


def _make_agg(d, tc_tiling=True):
    @functools.partial(
        pl.kernel,
        mesh=_make_mesh(),
        compiler_params=pltpu.CompilerParams(use_tc_tiling_on_sc=tc_tiling),
        out_type=jax.ShapeDtypeStruct((_NC, _NA, d), jnp.float32),
        scratch_types=[
            pltpu.VMEM((_IDXH, _CHUNK), jnp.int32),
            pltpu.VMEM((_IDXH, _CHUNK), jnp.int32),
            pltpu.VMEM((_NBUF, _CHUNK, d), jnp.float32),
            pltpu.VMEM_SHARED((_NA, d), jnp.float32),
        ]
        + [pltpu.SemaphoreType.DMA] * (2 * _NBUF),
    )
    def agg(t_hbm, srcp_hbm, dstp_hbm, out_hbm,
            src_all, dst_all, bufs, acc, *sems):
        gsems = sems[:_NBUF]
        ssems = sems[_NBUF:]
        c = lax.axis_index("c")
        s = lax.axis_index("s")
        wid = c * _NS + s
        r0 = s * _RPS

        def zbody(k, carry):
            for q in range(d // 16):
                bufs[0, k, pl.ds(q * 16, 16)] = jnp.zeros((16,), jnp.float32)
            return carry

        lax.fori_loop(0, _CHUNK, zbody, 0)
        for blk in range(_RPS // _CHUNK):
            pltpu.sync_copy(bufs.at[0],
                            acc.at[pl.ds(r0 + blk * _CHUNK, _CHUNK), :])
        rem = _RPS % _CHUNK
        pltpu.sync_copy(bufs.at[0, pl.ds(0, rem)],
                        acc.at[pl.ds(r0 + _RPS - rem, rem), :])
        plsc.subcore_barrier()

        def body(g, carry):
            j0 = g * _NBUF
            handles = []
            for b in range(_NBUF):
                @pl.when(g > 0)
                def _(b=b):
                    pltpu.make_async_copy(
                        bufs.at[b], acc.at[dst_all.at[j0 - _NBUF + b]], ssems[b]
                    ).wait()
                handles.append(
                    pltpu.async_copy(t_hbm.at[src_all.at[j0 + b]], bufs.at[b], gsems[b])
                )
            for b in range(_NBUF):
                handles[b].wait()
                pltpu.async_copy(bufs.at[b], acc.at[dst_all.at[j0 + b]], ssems[b],
                                 add=True)
            return carry

        ngrp = _IDXH // _NBUF
        for h in range(_NCH // _IDXH):
            pltpu.sync_copy(srcp_hbm.at[wid, pl.ds(h * _IDXH, _IDXH)], src_all)
            pltpu.sync_copy(dstp_hbm.at[wid, pl.ds(h * _IDXH, _IDXH)], dst_all)
            lax.fori_loop(0, ngrp, body, 0)
            for b in range(_NBUF):
                pltpu.make_async_copy(
                    bufs.at[b], acc.at[dst_all.at[(ngrp - 1) * _NBUF + b]], ssems[b]
                ).wait()

        plsc.subcore_barrier()
        pltpu.sync_copy(acc.at[pl.ds(r0, _RPS), :], out_hbm.at[c, pl.ds(r0, _RPS), :])

    return agg


_agg128 = _make_agg(128)
_agg64 = _make_agg(64, tc_tiling=False)


def _mm_xw(x, w):
    def body(x_ref, w_ref, o_ref):
        o_ref[...] = jnp.dot(x_ref[...], w_ref[...],
                             preferred_element_type=jnp.float32)

    din, dout = w.shape
    return pl.pallas_call(
        body,
        grid=(_N // _ROW_BLK,),
        in_specs=[
            pl.BlockSpec((_ROW_BLK, din), lambda i: (i, 0)),
            pl.BlockSpec((din, dout), lambda i: (0, 0)),
        ],
        out_specs=pl.BlockSpec((_ROW_BLK, dout), lambda i: (i, 0)),
        out_shape=jax.ShapeDtypeStruct((_N, dout), jnp.float32),
    )(x, w)


def _mm_scale(xw, deg_p):
    def body(x_ref, deg_ref, t_ref, d_ref):
        deg = deg_ref[0] + deg_ref[1]
        dn = jnp.where(deg > 0, lax.rsqrt(jnp.maximum(deg, 1.0)), 0.0)
        t_ref[...] = x_ref[...] * dn
        d_ref[...] = dn

    dout = xw.shape[-1]
    return pl.pallas_call(
        body,
        grid=(_N // _ROW_BLK,),
        in_specs=[
            pl.BlockSpec((_ROW_BLK, dout), lambda i: (i, 0)),
            pl.BlockSpec((2, _ROW_BLK, 1), lambda i: (0, i, 0)),
        ],
        out_specs=[
            pl.BlockSpec((_ROW_BLK, dout), lambda i: (i, 0)),
            pl.BlockSpec((_ROW_BLK, 1), lambda i: (i, 0)),
        ],
        out_shape=[
            jax.ShapeDtypeStruct((_N, dout), jnp.float32),
            jax.ShapeDtypeStruct((_N, 1), jnp.float32),
        ],
    )(xw, deg_p)


def _mm_mid(agg_p, d, b, w):
    def body(a_ref, d_ref, b_ref, w_ref, o_ref):
        dn = d_ref[...]
        h = jnp.maximum((a_ref[0] + a_ref[1]) * dn + b_ref[...], 0.0) * dn
        o_ref[...] = jnp.dot(h, w_ref[...], preferred_element_type=jnp.float32)

    din, dout = w.shape
    return pl.pallas_call(
        body,
        grid=(_N // _ROW_BLK,),
        in_specs=[
            pl.BlockSpec((2, _ROW_BLK, din), lambda i: (0, i, 0)),
            pl.BlockSpec((_ROW_BLK, 1), lambda i: (i, 0)),
            pl.BlockSpec((1, din), lambda i: (0, 0)),
            pl.BlockSpec((din, dout), lambda i: (0, 0)),
        ],
        out_specs=pl.BlockSpec((_ROW_BLK, dout), lambda i: (i, 0)),
        out_shape=jax.ShapeDtypeStruct((_N, dout), jnp.float32),
    )(agg_p, d, b, w)


def _mm_last(agg_p, d, b):
    def body(a_ref, d_ref, b_ref, o_ref):
        o_ref[...] = (a_ref[0] + a_ref[1]) * d_ref[...] + b_ref[...]

    dout = agg_p.shape[-1]
    return pl.pallas_call(
        body,
        grid=(_N // _ROW_BLK,),
        in_specs=[
            pl.BlockSpec((2, _ROW_BLK, dout), lambda i: (0, i, 0)),
            pl.BlockSpec((_ROW_BLK, 1), lambda i: (i, 0)),
            pl.BlockSpec((1, dout), lambda i: (0, 0)),
        ],
        out_specs=pl.BlockSpec((_ROW_BLK, dout), lambda i: (i, 0)),
        out_shape=jax.ShapeDtypeStruct((_N, dout), jnp.float32),
    )(agg_p, d, b)


def kernel(features, edge_index, W1, b1, W2, b2, W3, b3):
    pad = _NW * _EPWP - _E
    pad_src = (jnp.arange(pad, dtype=jnp.int32) * 97) % _N
    pad_dst = _N + (jnp.arange(pad, dtype=jnp.int32) % (_NA - _N))
    srcp = jnp.concatenate([edge_index[0], pad_src]).reshape(_NW, _NCH, _CHUNK)
    dstp = jnp.concatenate([edge_index[1], pad_dst]).reshape(_NW, _NCH, _CHUNK)
    xw = _mm_xw(features, W1)
    deg_p = _deg_sc(dstp).reshape(_NC, _NP, 1)
    t1, d = _mm_scale(xw, deg_p)
    agg1 = _agg128(t1, srcp, dstp)
    t2 = _mm_mid(agg1, d, b1.reshape(1, -1), W2)
    agg2 = _agg128(t2, srcp, dstp)
    t3 = _mm_mid(agg2, d, b2.reshape(1, -1), W3)
    agg3 = _agg64(t3, srcp, dstp)
    out = _mm_last(agg3, d, b3.reshape(1, -1))
    return out


# speedup vs baseline: 1.0021x; 1.0021x over previous
import functools

import jax
import jax.numpy as jnp
from jax import lax
from jax.experimental import pallas as pl
from jax.experimental.pallas import tpu as pltpu
from jax.experimental.pallas import tpu_sc as plsc

_N = 10000
_E = 320000
_NC = 2
_NS = 16
_NW = _NC * _NS
_EPW = _E // _NW
_CHUNK = 128
_NCH = 80
_EPWP = _CHUNK * _NCH
_NBUF = 2
_IDXH = 40
_NGRP = _NCH // _NBUF
_TRASH = 10100
_NA = _NS * 632
_RPS = _NA // _NS
_NP = _NS * 640
_DPS = _NP // _NS

_ROW_BLK = 1000


def _make_mesh():
    return plsc.VectorSubcoreMesh(core_axis_name="c", subcore_axis_name="s")


@functools.partial(
    pl.kernel,
    mesh=_make_mesh(),
    out_type=jax.ShapeDtypeStruct((_NC * _NP,), jnp.float32),
    scratch_types=[
        pltpu.VMEM((_NCH, _CHUNK), jnp.int32),
        pltpu.VMEM((_CHUNK,), jnp.float32),
        pltpu.VMEM((_DPS,), jnp.float32),
        pltpu.VMEM_SHARED((_NP,), jnp.float32),
    ],
)
def _deg_sc(dstp_hbm, out_hbm, dst_all, ones_v, stage_v, dacc):
    c = lax.axis_index("c")
    s = lax.axis_index("s")
    wid = c * _NS + s
    r0 = s * _DPS

    def zbody(k, carry):
        stage_v[pl.ds(k * 16, 16)] = jnp.zeros((16,), jnp.float32)
        return carry

    lax.fori_loop(0, _DPS // 16, zbody, 0)
    pltpu.sync_copy(stage_v, dacc.at[pl.ds(r0, _DPS)])
    for k in range(_CHUNK // 16):
        ones_v[pl.ds(k * 16, 16)] = jnp.full((16,), 1.0, jnp.float32)
    pltpu.sync_copy(dstp_hbm.at[wid], dst_all)
    plsc.subcore_barrier()

    def body(j, carry):
        pltpu.sync_copy(ones_v, dacc.at[dst_all.at[j]], add=True)
        return carry

    lax.fori_loop(0, _NCH, body, 0)
    plsc.subcore_barrier()
    pltpu.sync_copy(dacc.at[pl.ds(r0, _DPS)], stage_v)
    pltpu.sync_copy(stage_v, out_hbm.at[pl.ds(c * _NP + r0, _DPS)])


def _make_agg(d, tc_tiling=True):
    @functools.partial(
        pl.kernel,
        mesh=_make_mesh(),
        compiler_params=pltpu.CompilerParams(use_tc_tiling_on_sc=tc_tiling),
        out_type=jax.ShapeDtypeStruct((_NC, _NA, d), jnp.float32),
        scratch_types=[
            pltpu.VMEM((_IDXH, _CHUNK), jnp.int32),
            pltpu.VMEM((_IDXH, _CHUNK), jnp.int32),
            pltpu.VMEM((_NBUF, _CHUNK, d), jnp.float32),
            pltpu.VMEM_SHARED((_NA, d), jnp.float32),
        ]
        + [pltpu.SemaphoreType.DMA] * (2 * _NBUF),
    )
    def agg(t_hbm, srcp_hbm, dstp_hbm, out_hbm,
            src_all, dst_all, bufs, acc, *sems):
        gsems = sems[:_NBUF]
        ssems = sems[_NBUF:]
        c = lax.axis_index("c")
        s = lax.axis_index("s")
        wid = c * _NS + s
        r0 = s * _RPS

        def zbody(k, carry):
            for q in range(d // 16):
                bufs[0, k, pl.ds(q * 16, 16)] = jnp.zeros((16,), jnp.float32)
            return carry

        lax.fori_loop(0, _CHUNK, zbody, 0)
        for blk in range(_RPS // _CHUNK):
            pltpu.sync_copy(bufs.at[0],
                            acc.at[pl.ds(r0 + blk * _CHUNK, _CHUNK), :])
        rem = _RPS % _CHUNK
        pltpu.sync_copy(bufs.at[0, pl.ds(0, rem)],
                        acc.at[pl.ds(r0 + _RPS - rem, rem), :])
        plsc.subcore_barrier()

        def body(g, carry):
            j0 = g * _NBUF
            handles = []
            for b in range(_NBUF):
                @pl.when(g > 0)
                def _(b=b):
                    pltpu.make_async_copy(
                        bufs.at[b], acc.at[dst_all.at[j0 - _NBUF + b]], ssems[b]
                    ).wait()
                handles.append(
                    pltpu.async_copy(t_hbm.at[src_all.at[j0 + b]], bufs.at[b], gsems[b])
                )
            for b in range(_NBUF):
                handles[b].wait()
                pltpu.async_copy(bufs.at[b], acc.at[dst_all.at[j0 + b]], ssems[b],
                                 add=True)
            return carry

        ngrp = _IDXH // _NBUF
        for h in range(_NCH // _IDXH):
            pltpu.sync_copy(srcp_hbm.at[wid, pl.ds(h * _IDXH, _IDXH)], src_all)
            pltpu.sync_copy(dstp_hbm.at[wid, pl.ds(h * _IDXH, _IDXH)], dst_all)
            lax.fori_loop(0, ngrp, body, 0)
            for b in range(_NBUF):
                pltpu.make_async_copy(
                    bufs.at[b], acc.at[dst_all.at[(ngrp - 1) * _NBUF + b]], ssems[b]
                ).wait()

        plsc.subcore_barrier()
        pltpu.sync_copy(acc.at[pl.ds(r0, _RPS), :], out_hbm.at[c, pl.ds(r0, _RPS), :])

    return agg


_agg128 = _make_agg(128)
_agg64 = _make_agg(64, tc_tiling=False)


def _mm_first(x, deg_p, w):
    def body(x_ref, deg_ref, w_ref, t_ref, d_ref):
        deg = deg_ref[0] + deg_ref[1]
        dn = jnp.where(deg > 0, lax.rsqrt(jnp.maximum(deg, 1.0)), 0.0)
        t_ref[...] = jnp.dot(x_ref[...] * dn, w_ref[...],
                             preferred_element_type=jnp.float32)
        d_ref[...] = dn

    din, dout = w.shape
    return pl.pallas_call(
        body,
        grid=(_N // _ROW_BLK,),
        in_specs=[
            pl.BlockSpec((_ROW_BLK, din), lambda i: (i, 0)),
            pl.BlockSpec((2, _ROW_BLK, 1), lambda i: (0, i, 0)),
            pl.BlockSpec((din, dout), lambda i: (0, 0)),
        ],
        out_specs=[
            pl.BlockSpec((_ROW_BLK, dout), lambda i: (i, 0)),
            pl.BlockSpec((_ROW_BLK, 1), lambda i: (i, 0)),
        ],
        out_shape=[
            jax.ShapeDtypeStruct((_N, dout), jnp.float32),
            jax.ShapeDtypeStruct((_N, 1), jnp.float32),
        ],
    )(x, deg_p, w)


def _mm_mid(agg_p, d, b, w):
    def body(a_ref, d_ref, b_ref, w_ref, o_ref):
        dn = d_ref[...]
        h = jnp.maximum((a_ref[0] + a_ref[1]) * dn + b_ref[...], 0.0) * dn
        o_ref[...] = jnp.dot(h, w_ref[...], preferred_element_type=jnp.float32)

    din, dout = w.shape
    return pl.pallas_call(
        body,
        grid=(_N // _ROW_BLK,),
        in_specs=[
            pl.BlockSpec((2, _ROW_BLK, din), lambda i: (0, i, 0)),
            pl.BlockSpec((_ROW_BLK, 1), lambda i: (i, 0)),
            pl.BlockSpec((1, din), lambda i: (0, 0)),
            pl.BlockSpec((din, dout), lambda i: (0, 0)),
        ],
        out_specs=pl.BlockSpec((_ROW_BLK, dout), lambda i: (i, 0)),
        out_shape=jax.ShapeDtypeStruct((_N, dout), jnp.float32),
    )(agg_p, d, b, w)


def _mm_last(agg_p, d, b):
    def body(a_ref, d_ref, b_ref, o_ref):
        o_ref[...] = (a_ref[0] + a_ref[1]) * d_ref[...] + b_ref[...]

    dout = agg_p.shape[-1]
    return pl.pallas_call(
        body,
        grid=(_N // _ROW_BLK,),
        in_specs=[
            pl.BlockSpec((2, _ROW_BLK, dout), lambda i: (0, i, 0)),
            pl.BlockSpec((_ROW_BLK, 1), lambda i: (i, 0)),
            pl.BlockSpec((1, dout), lambda i: (0, 0)),
        ],
        out_specs=pl.BlockSpec((_ROW_BLK, dout), lambda i: (i, 0)),
        out_shape=jax.ShapeDtypeStruct((_N, dout), jnp.float32),
    )(agg_p, d, b)


def kernel(features, edge_index, W1, b1, W2, b2, W3, b3):
    pad = _NW * _EPWP - _E
    pad_src = (jnp.arange(pad, dtype=jnp.int32) * 97) % _N
    pad_dst = _N + (jnp.arange(pad, dtype=jnp.int32) % (_NA - _N))
    srcp = jnp.concatenate([edge_index[0], pad_src]).reshape(_NW, _NCH, _CHUNK)
    dstp = jnp.concatenate([edge_index[1], pad_dst]).reshape(_NW, _NCH, _CHUNK)
    deg_p = _deg_sc(dstp).reshape(_NC, _NP, 1)
    t1, d = _mm_first(features, deg_p, W1)
    agg1 = _agg128(t1, srcp, dstp)
    t2 = _mm_mid(agg1, d, b1.reshape(1, -1), W2)
    agg2 = _agg128(t2, srcp, dstp)
    t3 = _mm_mid(agg2, d, b2.reshape(1, -1), W3)
    agg3 = _agg64(t3, srcp, dstp)
    out = _mm_last(agg3, d, b3.reshape(1, -1))
    return out


# speedup vs baseline: 1.0023x; 1.0002x over previous
import functools

import jax
import jax.numpy as jnp
from jax import lax
from jax.experimental import pallas as pl
from jax.experimental.pallas import tpu as pltpu
from jax.experimental.pallas import tpu_sc as plsc

_N = 10000
_E = 320000
_NC = 2
_NS = 16
_NW = _NC * _NS
_CHUNK = 128
_NCH = 80
_EPWP = _CHUNK * _NCH
_NBUF = 2
_IDXH = 40
_NA = _NS * 632
_RPS = _NA // _NS
_NP = _NS * 640
_DPS = _NP // _NS

_ROW_BLK = 1000


def _make_mesh():
    return plsc.VectorSubcoreMesh(core_axis_name="c", subcore_axis_name="s")


@functools.partial(
    pl.kernel,
    mesh=_make_mesh(),
    out_type=jax.ShapeDtypeStruct((_NC * _NP,), jnp.float32),
    scratch_types=[
        pltpu.VMEM((_NCH, _CHUNK), jnp.int32),
        pltpu.VMEM((_CHUNK,), jnp.float32),
        pltpu.VMEM((_DPS,), jnp.float32),
        pltpu.VMEM_SHARED((_NP,), jnp.float32),
    ],
)
def _deg_sc(dstp_hbm, out_hbm, dst_all, ones_v, stage_v, dacc):
    c = lax.axis_index("c")
    s = lax.axis_index("s")
    wid = c * _NS + s
    r0 = s * _DPS

    def zbody(k, carry):
        stage_v[pl.ds(k * 16, 16)] = jnp.zeros((16,), jnp.float32)
        return carry

    lax.fori_loop(0, _DPS // 16, zbody, 0)
    pltpu.sync_copy(stage_v, dacc.at[pl.ds(r0, _DPS)])
    for k in range(_CHUNK // 16):
        ones_v[pl.ds(k * 16, 16)] = jnp.full((16,), 1.0, jnp.float32)
    pltpu.sync_copy(dstp_hbm.at[wid], dst_all)
    plsc.subcore_barrier()

    def body(j, carry):
        pltpu.sync_copy(ones_v, dacc.at[dst_all.at[j]], add=True)
        return carry

    lax.fori_loop(0, _NCH, body, 0)
    plsc.subcore_barrier()
    pltpu.sync_copy(dacc.at[pl.ds(r0, _DPS)], stage_v)
    pltpu.sync_copy(stage_v, out_hbm.at[pl.ds(c * _NP + r0, _DPS)])


def _make_agg(d, tc_tiling=True):
    @functools.partial(
        pl.kernel,
        mesh=_make_mesh(),
        compiler_params=pltpu.CompilerParams(use_tc_tiling_on_sc=tc_tiling),
        out_type=jax.ShapeDtypeStruct((_NC, _NA, d), jnp.float32),
        scratch_types=[
            pltpu.VMEM((_IDXH, _CHUNK), jnp.int32),
            pltpu.VMEM((_IDXH, _CHUNK), jnp.int32),
            pltpu.VMEM((_NBUF, _CHUNK, d), jnp.float32),
            pltpu.VMEM_SHARED((_NA, d), jnp.float32),
        ]
        + [pltpu.SemaphoreType.DMA] * (2 * _NBUF),
    )
    def agg(t_hbm, srcp_hbm, dstp_hbm, out_hbm,
            src_all, dst_all, bufs, acc, *sems):
        gsems = sems[:_NBUF]
        ssems = sems[_NBUF:]
        c = lax.axis_index("c")
        s = lax.axis_index("s")
        wid = c * _NS + s
        r0 = s * _RPS

        def zbody(k, carry):
            for q in range(d // 16):
                bufs[0, k, pl.ds(q * 16, 16)] = jnp.zeros((16,), jnp.float32)
            return carry

        lax.fori_loop(0, _CHUNK, zbody, 0)
        for blk in range(_RPS // _CHUNK):
            pltpu.sync_copy(bufs.at[0],
                            acc.at[pl.ds(r0 + blk * _CHUNK, _CHUNK), :])
        rem = _RPS % _CHUNK
        pltpu.sync_copy(bufs.at[0, pl.ds(0, rem)],
                        acc.at[pl.ds(r0 + _RPS - rem, rem), :])
        plsc.subcore_barrier()

        def body(g, carry):
            j0 = g * _NBUF
            handles = []
            for b in range(_NBUF):
                @pl.when(g > 0)
                def _(b=b):
                    pltpu.make_async_copy(
                        bufs.at[b], acc.at[dst_all.at[j0 - _NBUF + b]], ssems[b]
                    ).wait()
                handles.append(
                    pltpu.async_copy(t_hbm.at[src_all.at[j0 + b]], bufs.at[b], gsems[b])
                )
            for b in range(_NBUF):
                handles[b].wait()
                pltpu.async_copy(bufs.at[b], acc.at[dst_all.at[j0 + b]], ssems[b],
                                 add=True)
            return carry

        ngrp = _IDXH // _NBUF
        for h in range(_NCH // _IDXH):
            pltpu.sync_copy(srcp_hbm.at[wid, pl.ds(h * _IDXH, _IDXH)], src_all)
            pltpu.sync_copy(dstp_hbm.at[wid, pl.ds(h * _IDXH, _IDXH)], dst_all)
            lax.fori_loop(0, ngrp, body, 0)
            for b in range(_NBUF):
                pltpu.make_async_copy(
                    bufs.at[b], acc.at[dst_all.at[(ngrp - 1) * _NBUF + b]], ssems[b]
                ).wait()

        plsc.subcore_barrier()
        pltpu.sync_copy(acc.at[pl.ds(r0, _RPS), :], out_hbm.at[c, pl.ds(r0, _RPS), :])

    return agg


_agg128 = _make_agg(128)
_agg64 = _make_agg(64, tc_tiling=False)


def _mm_first(x, deg_p, w):
    def body(x_ref, deg_ref, w_ref, t_ref, d_ref):
        deg = deg_ref[0] + deg_ref[1]
        dn = jnp.where(deg > 0, lax.rsqrt(jnp.maximum(deg, 1.0)), 0.0)
        t_ref[...] = jnp.dot(x_ref[...] * dn, w_ref[...],
                             preferred_element_type=jnp.float32)
        d_ref[...] = dn

    din, dout = w.shape
    return pl.pallas_call(
        body,
        grid=(_N // _ROW_BLK,),
        in_specs=[
            pl.BlockSpec((_ROW_BLK, din), lambda i: (i, 0)),
            pl.BlockSpec((2, _ROW_BLK, 1), lambda i: (0, i, 0)),
            pl.BlockSpec((din, dout), lambda i: (0, 0)),
        ],
        out_specs=[
            pl.BlockSpec((_ROW_BLK, dout), lambda i: (i, 0)),
            pl.BlockSpec((_ROW_BLK, 1), lambda i: (i, 0)),
        ],
        out_shape=[
            jax.ShapeDtypeStruct((_N, dout), jnp.float32),
            jax.ShapeDtypeStruct((_N, 1), jnp.float32),
        ],
    )(x, deg_p, w)


def _mm_mid(agg_p, d, b, w):
    def body(a_ref, d_ref, b_ref, w_ref, o_ref):
        dn = d_ref[...]
        h = jnp.maximum((a_ref[0] + a_ref[1]) * dn + b_ref[...], 0.0) * dn
        o_ref[...] = jnp.dot(h, w_ref[...], preferred_element_type=jnp.float32)

    din, dout = w.shape
    return pl.pallas_call(
        body,
        grid=(_N // _ROW_BLK,),
        in_specs=[
            pl.BlockSpec((2, _ROW_BLK, din), lambda i: (0, i, 0)),
            pl.BlockSpec((_ROW_BLK, 1), lambda i: (i, 0)),
            pl.BlockSpec((1, din), lambda i: (0, 0)),
            pl.BlockSpec((din, dout), lambda i: (0, 0)),
        ],
        out_specs=pl.BlockSpec((_ROW_BLK, dout), lambda i: (i, 0)),
        out_shape=jax.ShapeDtypeStruct((_N, dout), jnp.float32),
    )(agg_p, d, b, w)


def _mm_last(agg_p, d, b):
    def body(a_ref, d_ref, b_ref, o_ref):
        o_ref[...] = (a_ref[0] + a_ref[1]) * d_ref[...] + b_ref[...]

    dout = agg_p.shape[-1]
    return pl.pallas_call(
        body,
        grid=(_N // _ROW_BLK,),
        in_specs=[
            pl.BlockSpec((2, _ROW_BLK, dout), lambda i: (0, i, 0)),
            pl.BlockSpec((_ROW_BLK, 1), lambda i: (i, 0)),
            pl.BlockSpec((1, dout), lambda i: (0, 0)),
        ],
        out_specs=pl.BlockSpec((_ROW_BLK, dout), lambda i: (i, 0)),
        out_shape=jax.ShapeDtypeStruct((_N, dout), jnp.float32),
    )(agg_p, d, b)


def kernel(features, edge_index, W1, b1, W2, b2, W3, b3):
    pad = _NW * _EPWP - _E
    pad_src = (jnp.arange(pad, dtype=jnp.int32) * 97) % _N
    pad_dst = _N + (jnp.arange(pad, dtype=jnp.int32) % (_NA - _N))
    srcp = jnp.concatenate([edge_index[0], pad_src]).reshape(_NW, _NCH, _CHUNK)
    dstp = jnp.concatenate([edge_index[1], pad_dst]).reshape(_NW, _NCH, _CHUNK)
    deg_p = _deg_sc(dstp).reshape(_NC, _NP, 1)
    t1, d = _mm_first(features, deg_p, W1)
    agg1 = _agg128(t1, srcp, dstp)
    t2 = _mm_mid(agg1, d, b1.reshape(1, -1), W2)
    agg2 = _agg128(t2, srcp, dstp)
    t3 = _mm_mid(agg2, d, b2.reshape(1, -1), W3)
    agg3 = _agg64(t3, srcp, dstp)
    out = _mm_last(agg3, d, b3.reshape(1, -1))
    return out


# speedup vs baseline: 1.0071x; 1.0048x over previous
import functools

import jax
import jax.numpy as jnp
from jax import lax
from jax.experimental import pallas as pl
from jax.experimental.pallas import tpu as pltpu
from jax.experimental.pallas import tpu_sc as plsc

_N = 10000
_E = 320000
_NC = 2
_NS = 16
_NW = _NC * _NS
_CHUNK = 128
_NCH = 80
_EPWP = _CHUNK * _NCH
_NBUF = 2
_IDXH = 40
_NA = _NS * 632
_RPS = _NA // _NS
_NP = _NS * 640
_DPS = _NP // _NS

_ROW_BLK = 1000


def _make_mesh():
    return plsc.VectorSubcoreMesh(core_axis_name="c", subcore_axis_name="s")


@functools.partial(
    pl.kernel,
    mesh=_make_mesh(),
    out_type=jax.ShapeDtypeStruct((_NC * _NP,), jnp.float32),
    scratch_types=[
        pltpu.VMEM((_NCH, _CHUNK), jnp.int32),
        pltpu.VMEM((_CHUNK,), jnp.float32),
        pltpu.VMEM((_DPS,), jnp.float32),
        pltpu.VMEM_SHARED((_NP,), jnp.float32),
        pltpu.SemaphoreType.DMA,
        pltpu.SemaphoreType.DMA,
    ],
)
def _deg_sc(dstp_hbm, out_hbm, dst_all, ones_v, stage_v, dacc, sem0, sem1):
    c = lax.axis_index("c")
    s = lax.axis_index("s")
    wid = c * _NS + s
    r0 = s * _DPS

    def zbody(k, carry):
        stage_v[pl.ds(k * 16, 16)] = jnp.zeros((16,), jnp.float32)
        return carry

    lax.fori_loop(0, _DPS // 16, zbody, 0)
    pltpu.sync_copy(stage_v, dacc.at[pl.ds(r0, _DPS)])
    for k in range(_CHUNK // 16):
        ones_v[pl.ds(k * 16, 16)] = jnp.full((16,), 1.0, jnp.float32)
    pltpu.sync_copy(dstp_hbm.at[wid], dst_all)
    plsc.subcore_barrier()

    dsems = (sem0, sem1)

    def body(g, carry):
        j0 = g * 2
        for b in range(2):
            @pl.when(g > 0)
            def _(b=b):
                pltpu.make_async_copy(
                    ones_v, dacc.at[dst_all.at[j0 - 2 + b]], dsems[b]
                ).wait()
            pltpu.async_copy(ones_v, dacc.at[dst_all.at[j0 + b]], dsems[b],
                             add=True)
        return carry

    lax.fori_loop(0, _NCH // 2, body, 0)
    for b in range(2):
        pltpu.make_async_copy(
            ones_v, dacc.at[dst_all.at[_NCH - 2 + b]], dsems[b]
        ).wait()
    plsc.subcore_barrier()
    pltpu.sync_copy(dacc.at[pl.ds(r0, _DPS)], stage_v)
    pltpu.sync_copy(stage_v, out_hbm.at[pl.ds(c * _NP + r0, _DPS)])


def _make_agg(d, tc_tiling=True):
    @functools.partial(
        pl.kernel,
        mesh=_make_mesh(),
        compiler_params=pltpu.CompilerParams(use_tc_tiling_on_sc=tc_tiling),
        out_type=jax.ShapeDtypeStruct((_NC, _NA, d), jnp.float32),
        scratch_types=[
            pltpu.VMEM((_IDXH, _CHUNK), jnp.int32),
            pltpu.VMEM((_IDXH, _CHUNK), jnp.int32),
            pltpu.VMEM((_NBUF, _CHUNK, d), jnp.float32),
            pltpu.VMEM_SHARED((_NA, d), jnp.float32),
        ]
        + [pltpu.SemaphoreType.DMA] * (2 * _NBUF),
    )
    def agg(t_hbm, srcp_hbm, dstp_hbm, out_hbm,
            src_all, dst_all, bufs, acc, *sems):
        gsems = sems[:_NBUF]
        ssems = sems[_NBUF:]
        c = lax.axis_index("c")
        s = lax.axis_index("s")
        wid = c * _NS + s
        r0 = s * _RPS

        def zbody(k, carry):
            for q in range(d // 16):
                bufs[0, k, pl.ds(q * 16, 16)] = jnp.zeros((16,), jnp.float32)
            return carry

        lax.fori_loop(0, _CHUNK, zbody, 0)
        for blk in range(_RPS // _CHUNK):
            pltpu.sync_copy(bufs.at[0],
                            acc.at[pl.ds(r0 + blk * _CHUNK, _CHUNK), :])
        rem = _RPS % _CHUNK
        pltpu.sync_copy(bufs.at[0, pl.ds(0, rem)],
                        acc.at[pl.ds(r0 + _RPS - rem, rem), :])
        plsc.subcore_barrier()

        def body(g, carry):
            j0 = g * _NBUF
            handles = []
            for b in range(_NBUF):
                @pl.when(g > 0)
                def _(b=b):
                    pltpu.make_async_copy(
                        bufs.at[b], acc.at[dst_all.at[j0 - _NBUF + b]], ssems[b]
                    ).wait()
                handles.append(
                    pltpu.async_copy(t_hbm.at[src_all.at[j0 + b]], bufs.at[b], gsems[b])
                )
            for b in range(_NBUF):
                handles[b].wait()
                pltpu.async_copy(bufs.at[b], acc.at[dst_all.at[j0 + b]], ssems[b],
                                 add=True)
            return carry

        ngrp = _IDXH // _NBUF
        for h in range(_NCH // _IDXH):
            pltpu.sync_copy(srcp_hbm.at[wid, pl.ds(h * _IDXH, _IDXH)], src_all)
            pltpu.sync_copy(dstp_hbm.at[wid, pl.ds(h * _IDXH, _IDXH)], dst_all)
            lax.fori_loop(0, ngrp, body, 0)
            for b in range(_NBUF):
                pltpu.make_async_copy(
                    bufs.at[b], acc.at[dst_all.at[(ngrp - 1) * _NBUF + b]], ssems[b]
                ).wait()

        plsc.subcore_barrier()
        pltpu.sync_copy(acc.at[pl.ds(r0, _RPS), :], out_hbm.at[c, pl.ds(r0, _RPS), :])

    return agg


_agg128 = _make_agg(128)
_agg64 = _make_agg(64, tc_tiling=False)


def _mm_first(x, deg_p, w):
    def body(x_ref, deg_ref, w_ref, t_ref, d_ref):
        deg = deg_ref[0] + deg_ref[1]
        dn = jnp.where(deg > 0, lax.rsqrt(jnp.maximum(deg, 1.0)), 0.0)
        t_ref[...] = jnp.dot(x_ref[...] * dn, w_ref[...],
                             preferred_element_type=jnp.float32)
        d_ref[...] = dn

    din, dout = w.shape
    return pl.pallas_call(
        body,
        grid=(_N // _ROW_BLK,),
        in_specs=[
            pl.BlockSpec((_ROW_BLK, din), lambda i: (i, 0)),
            pl.BlockSpec((2, _ROW_BLK, 1), lambda i: (0, i, 0)),
            pl.BlockSpec((din, dout), lambda i: (0, 0)),
        ],
        out_specs=[
            pl.BlockSpec((_ROW_BLK, dout), lambda i: (i, 0)),
            pl.BlockSpec((_ROW_BLK, 1), lambda i: (i, 0)),
        ],
        out_shape=[
            jax.ShapeDtypeStruct((_N, dout), jnp.float32),
            jax.ShapeDtypeStruct((_N, 1), jnp.float32),
        ],
    )(x, deg_p, w)


def _mm_mid(agg_p, d, b, w):
    def body(a_ref, d_ref, b_ref, w_ref, o_ref):
        dn = d_ref[...]
        h = jnp.maximum((a_ref[0] + a_ref[1]) * dn + b_ref[...], 0.0) * dn
        o_ref[...] = jnp.dot(h, w_ref[...], preferred_element_type=jnp.float32)

    din, dout = w.shape
    return pl.pallas_call(
        body,
        grid=(_N // _ROW_BLK,),
        in_specs=[
            pl.BlockSpec((2, _ROW_BLK, din), lambda i: (0, i, 0)),
            pl.BlockSpec((_ROW_BLK, 1), lambda i: (i, 0)),
            pl.BlockSpec((1, din), lambda i: (0, 0)),
            pl.BlockSpec((din, dout), lambda i: (0, 0)),
        ],
        out_specs=pl.BlockSpec((_ROW_BLK, dout), lambda i: (i, 0)),
        out_shape=jax.ShapeDtypeStruct((_N, dout), jnp.float32),
    )(agg_p, d, b, w)


def _mm_last(agg_p, d, b):
    def body(a_ref, d_ref, b_ref, o_ref):
        o_ref[...] = (a_ref[0] + a_ref[1]) * d_ref[...] + b_ref[...]

    dout = agg_p.shape[-1]
    return pl.pallas_call(
        body,
        grid=(_N // _ROW_BLK,),
        in_specs=[
            pl.BlockSpec((2, _ROW_BLK, dout), lambda i: (0, i, 0)),
            pl.BlockSpec((_ROW_BLK, 1), lambda i: (i, 0)),
            pl.BlockSpec((1, dout), lambda i: (0, 0)),
        ],
        out_specs=pl.BlockSpec((_ROW_BLK, dout), lambda i: (i, 0)),
        out_shape=jax.ShapeDtypeStruct((_N, dout), jnp.float32),
    )(agg_p, d, b)


def kernel(features, edge_index, W1, b1, W2, b2, W3, b3):
    pad = _NW * _EPWP - _E
    pad_src = (jnp.arange(pad, dtype=jnp.int32) * 97) % _N
    pad_dst = _N + (jnp.arange(pad, dtype=jnp.int32) % (_NA - _N))
    srcp = jnp.concatenate([edge_index[0], pad_src]).reshape(_NW, _NCH, _CHUNK)
    dstp = jnp.concatenate([edge_index[1], pad_dst]).reshape(_NW, _NCH, _CHUNK)
    deg_p = _deg_sc(dstp).reshape(_NC, _NP, 1)
    t1, d = _mm_first(features, deg_p, W1)
    agg1 = _agg128(t1, srcp, dstp)
    t2 = _mm_mid(agg1, d, b1.reshape(1, -1), W2)
    agg2 = _agg128(t2, srcp, dstp)
    t3 = _mm_mid(agg2, d, b2.reshape(1, -1), W3)
    agg3 = _agg64(t3, srcp, dstp)
    out = _mm_last(agg3, d, b3.reshape(1, -1))
    return out
